# baseline (speedup 1.0000x reference)
import jax
import jax.numpy as jnp
from jax import lax
from jax.experimental import pallas as pl
from jax.experimental.pallas import tpu as pltpu
from jax.experimental.pallas import tpu_sc as plsc

BATCH = 16384
IN_COLS = 100
OUT_COLS = 380
NUM_NONCAT = 60
_CAT_GROUPS = ((4, 60, 20, 60), (8, 80, 10, 140), (16, 90, 10, 220))

NUM_WORKERS = 32
ROWS_PER_WORKER = BATCH // NUM_WORKERS
CHUNK_ROWS = 128
CHUNKS = ROWS_PER_WORKER // CHUNK_ROWS
BLOCKS = CHUNK_ROWS // 16


def _block_body(in_v, out_v, block):
    rows = block * 16 + lax.iota(jnp.int32, 16)
    for c in range(NUM_NONCAT):
        col = jnp.full((16,), c, jnp.int32)
        v = plsc.load_gather(in_v, [rows, col])
        plsc.store_scatter(out_v, [rows, col], v)
    one = jnp.full((16,), 1.0, jnp.float32)
    zero = jnp.zeros((16,), jnp.float32)
    for card, src0, nparams, out0 in _CAT_GROUPS:
        for j in range(nparams):
            src = jnp.full((16,), src0 + j, jnp.int32)
            v = plsc.load_gather(in_v, [rows, src])
            for c in range(card):
                oh = jnp.where(v == float(c), one, zero)
                dst = jnp.full((16,), out0 + card * j + c, jnp.int32)
                plsc.store_scatter(out_v, [rows, dst], oh)


def _sc_kernel(x_hbm, out_hbm, in_v, out_v):
    wid = lax.axis_index("s") * 2 + lax.axis_index("c")
    row0 = wid * ROWS_PER_WORKER

    @pl.loop(0, CHUNKS)
    def _chunk(chunk):
        base = row0 + chunk * CHUNK_ROWS
        pltpu.sync_copy(x_hbm.at[pl.ds(base, CHUNK_ROWS)], in_v)

        @pl.loop(0, BLOCKS)
        def _blk(block):
            _block_body(in_v, out_v, block)

        pltpu.sync_copy(out_v, out_hbm.at[pl.ds(base, CHUNK_ROWS)])


@jax.jit
def kernel(x):
    mesh = plsc.VectorSubcoreMesh(core_axis_name="c", subcore_axis_name="s")
    f = pl.kernel(
        _sc_kernel,
        out_type=jax.ShapeDtypeStruct((BATCH, OUT_COLS), jnp.float32),
        mesh=mesh,
        scratch_types=[
            pltpu.VMEM((CHUNK_ROWS, IN_COLS), jnp.float32),
            pltpu.VMEM((CHUNK_ROWS, OUT_COLS), jnp.float32),
        ],
        compiler_params=pltpu.CompilerParams(needs_layout_passes=False),
    )
    return f(x)
